# Initial kernel scaffold; baseline (speedup 1.0000x reference)
import jax
import jax.numpy as jnp
from jax.experimental import pallas as pl


def kernel(x, edge_index, batch, W1, b1, W2, b2, W3, b3, fc1_W, fc1_b, fc2_W, fc2_b):
    raise NotImplementedError("write your pallas kernel here")


# speedup vs baseline: 6.0224x; 6.0224x over previous
import functools

import jax
import jax.numpy as jnp
from jax import lax
from jax.experimental import pallas as pl
from jax.experimental.pallas import tpu as pltpu
from jax.experimental.pallas import tpu_sc as plsc

N = 100000
E = 1600000
G = 128
NP = 100352
BLK = 3584
NBLK = NP // BLK

NC = 2
NS = 16
NW = NC * NS
EROWS = E // 128
_MESH = dict(core_axis_name="c", subcore_axis_name="s",
             num_cores=NC, num_subcores=NS)


def _stage_a_body(degp_ref, x_ref, dis_ref, xs_ref):
    degp = degp_ref[...]
    deg = degp[0] + degp[1] + 1.0
    dis = lax.rsqrt(deg)
    dis_ref[...] = dis
    xs_ref[...] = dis * x_ref[...]


def _stage_a(degp, x_p):
    return pl.pallas_call(
        _stage_a_body,
        grid=(NBLK,),
        in_specs=[pl.BlockSpec((2, BLK, 1), lambda i: (0, i, 0)),
                  pl.BlockSpec((BLK, 1), lambda i: (i, 0))],
        out_specs=[pl.BlockSpec((BLK, 1), lambda i: (i, 0)),
                   pl.BlockSpec((BLK, 1), lambda i: (i, 0))],
        out_shape=[jax.ShapeDtypeStruct((NP, 1), jnp.float32),
                   jax.ShapeDtypeStruct((NP, 1), jnp.float32)],
    )(degp, x_p)


def _stage_b_body(dis_ref, sp_ref, xs_ref, w1_ref, b1_ref, w2_ref, out_ref):
    dis = dis_ref[...]
    sp = sp_ref[...]
    s = sp[0] + sp[1]
    t1 = (dis * (s + xs_ref[...])) * w1_ref[...] + b1_ref[...]
    h1 = jnp.maximum(t1, 0.0)
    xw2 = jnp.dot(h1, w2_ref[...], preferred_element_type=jnp.float32)
    out_ref[...] = xw2 * dis


def _stage_b(dis, sp, xs, W1, b1, W2):
    return pl.pallas_call(
        _stage_b_body,
        grid=(NBLK,),
        in_specs=[pl.BlockSpec((BLK, 1), lambda i: (i, 0)),
                  pl.BlockSpec((2, BLK, 1), lambda i: (0, i, 0)),
                  pl.BlockSpec((BLK, 1), lambda i: (i, 0)),
                  pl.BlockSpec((1, 128), lambda i: (0, 0)),
                  pl.BlockSpec((1, 128), lambda i: (0, 0)),
                  pl.BlockSpec((128, 128), lambda i: (0, 0))],
        out_specs=pl.BlockSpec((BLK, 128), lambda i: (i, 0)),
        out_shape=jax.ShapeDtypeStruct((NP, 128), jnp.float32),
    )(dis, sp, xs, W1, b1, W2)


def _stage_c_body(dis_ref, agg_ref, xwp_ref, b_ref, w_ref, o0_ref, o1_ref):
    dis = dis_ref[...]
    h = jnp.maximum(dis * (agg_ref[...] + xwp_ref[...]) + b_ref[...], 0.0)
    xw = jnp.dot(h, w_ref[...], preferred_element_type=jnp.float32)
    xw = xw * dis
    o0_ref[...] = xw[:, :128]
    o1_ref[...] = xw[:, 128:]


def _stage_c(dis, agg2, xw2p, b2, W3):
    return pl.pallas_call(
        _stage_c_body,
        grid=(NBLK,),
        in_specs=[pl.BlockSpec((BLK, 1), lambda i: (i, 0)),
                  pl.BlockSpec((BLK, 128), lambda i: (i, 0)),
                  pl.BlockSpec((BLK, 128), lambda i: (i, 0)),
                  pl.BlockSpec((1, 128), lambda i: (0, 0)),
                  pl.BlockSpec((128, 256), lambda i: (0, 0))],
        out_specs=[pl.BlockSpec((BLK, 128), lambda i: (i, 0)),
                   pl.BlockSpec((BLK, 128), lambda i: (i, 0))],
        out_shape=[jax.ShapeDtypeStruct((NP, 128), jnp.float32),
                   jax.ShapeDtypeStruct((NP, 128), jnp.float32)],
    )(dis, agg2, xw2p, b2, W3)


def _stage_d_body(dis_ref, a0_ref, a1_ref, x0_ref, x1_ref, b3_ref, batch_ref,
                  fc1w_ref, fc1b_ref, fc2w_ref, fc2b_ref, out_ref, acc_ref):
    i = pl.program_id(0)

    @pl.when(i == 0)
    def _init():
        acc_ref[...] = jnp.zeros_like(acc_ref)

    dis = dis_ref[...]
    agg = jnp.concatenate([a0_ref[...], a1_ref[...]], axis=1)
    xwp = jnp.concatenate([x0_ref[...], x1_ref[...]], axis=1)
    h3 = jnp.maximum(dis * (agg + xwp) + b3_ref[...], 0.0)
    bvec = batch_ref[...]
    g_lo = bvec[0, 0]
    g_hi = bvec[BLK - 1, 0]
    gids = lax.broadcasted_iota(jnp.int32, (G, 1), 0)

    def seg_body(g, _):
        m = bvec == g
        contrib = jnp.max(jnp.where(m, h3, 0.0), axis=0, keepdims=True)
        acc_ref[...] = jnp.maximum(acc_ref[...],
                                   jnp.where(gids == g, contrib, 0.0))
        return 0

    lax.fori_loop(g_lo, g_hi + 1, seg_body, 0)

    @pl.when(i == NBLK - 1)
    def _fin():
        p = acc_ref[...]
        z = jnp.maximum(
            jnp.dot(p, fc1w_ref[...], preferred_element_type=jnp.float32)
            + fc1b_ref[...], 0.0)
        out_ref[...] = jnp.maximum(
            jnp.dot(z, fc2w_ref[...], preferred_element_type=jnp.float32)
            + fc2b_ref[...], 0.0)


def _stage_d(dis, a0, a1, x0, x1, b3, batch_p, fc1_W, fc1_b, fc2_W, fc2_b):
    return pl.pallas_call(
        _stage_d_body,
        grid=(NBLK,),
        in_specs=[pl.BlockSpec((BLK, 1), lambda i: (i, 0)),
                  pl.BlockSpec((BLK, 128), lambda i: (i, 0)),
                  pl.BlockSpec((BLK, 128), lambda i: (i, 0)),
                  pl.BlockSpec((BLK, 128), lambda i: (i, 0)),
                  pl.BlockSpec((BLK, 128), lambda i: (i, 0)),
                  pl.BlockSpec((1, 256), lambda i: (0, 0)),
                  pl.BlockSpec((BLK, 1), lambda i: (i, 0)),
                  pl.BlockSpec((256, 128), lambda i: (0, 0)),
                  pl.BlockSpec((1, 128), lambda i: (0, 0)),
                  pl.BlockSpec((128, 10), lambda i: (0, 0)),
                  pl.BlockSpec((1, 10), lambda i: (0, 0))],
        out_specs=pl.BlockSpec((G, 10), lambda i: (0, 0)),
        out_shape=jax.ShapeDtypeStruct((G, 10), jnp.float32),
        scratch_shapes=[pltpu.VMEM((G, 256), jnp.float32)],
    )(dis, a0, a1, x0, x1, b3, batch_p, fc1_W, fc1_b, fc2_W, fc2_b)


_NG = EROWS // 4
_GBASE, _GREM = _NG // NW, _NG % NW
_NG_SC = _NG
_TBASE, _TREM = _NG // NS, _NG % NS

_SLC = NP // NS


def _zero_vmem_rows(buf, rows, f):
    zero = jnp.zeros((16,), jnp.float32)
    for r in range(rows):
        for q in range(f // 16):
            buf[r, pl.ds(q * 16, 16)] = zero


def _deg_body(dst2d, out, acc, idx_v, ones_v, zbuf):
    cid = lax.axis_index("c")
    sid = lax.axis_index("s")
    wid = sid * NC + cid
    for q in range(8):
        zbuf[pl.ds(q * 16, 16)] = jnp.zeros((16,), jnp.float32)
        ones_v[pl.ds(q * 16, 16)] = jnp.ones((16,), jnp.float32)
    for q in range(_SLC // 128):
        pltpu.sync_copy(zbuf, acc.at[pl.ds(sid * _SLC + q * 128, 128)])
    plsc.subcore_barrier()
    gstart = wid * _GBASE + jnp.minimum(wid, _GREM)
    ng = _GBASE + (wid < _GREM).astype(jnp.int32)

    def grp(g, _):
        r0 = (gstart + g) * 4
        pltpu.sync_copy(dst2d.at[pl.ds(r0, 4)], idx_v)
        for j in range(4):
            pltpu.sync_copy(ones_v, acc.at[idx_v.at[j]], add=True)
        return 0

    lax.fori_loop(0, ng, grp, 0)
    plsc.subcore_barrier()
    pltpu.sync_copy(acc.at[pl.ds(sid * _SLC, _SLC)],
                    out.at[cid, pl.ds(sid * _SLC, _SLC)])


def _deg_scatter(dst2d):
    f = pl.kernel(
        _deg_body,
        out_type=jax.ShapeDtypeStruct((NC, NP), jnp.float32),
        mesh=plsc.VectorSubcoreMesh(**_MESH),
        scratch_types=[
            pltpu.VMEM_SHARED((NP,), jnp.float32),
            pltpu.VMEM((4, 128), jnp.int32),
            pltpu.VMEM((128,), jnp.float32),
            pltpu.VMEM((128,), jnp.float32),
        ],
    )
    return f(dst2d)


def _s_body(src2d, dst2d, xs_h, out, acc, idxs_v, idxd_v, vals_v, zbuf, sem):
    cid = lax.axis_index("c")
    sid = lax.axis_index("s")
    wid = sid * NC + cid
    for q in range(8):
        zbuf[pl.ds(q * 16, 16)] = jnp.zeros((16,), jnp.float32)
    for q in range(_SLC // 128):
        pltpu.sync_copy(zbuf, acc.at[pl.ds(sid * _SLC + q * 128, 128)])
    plsc.subcore_barrier()
    gstart = wid * _GBASE + jnp.minimum(wid, _GREM)
    ng = _GBASE + (wid < _GREM).astype(jnp.int32)

    def grp(g, _):
        r0 = (gstart + g) * 4
        pltpu.sync_copy(src2d.at[pl.ds(r0, 4)], idxs_v)
        pltpu.sync_copy(dst2d.at[pl.ds(r0, 4)], idxd_v)
        for j in range(4):
            pltpu.async_copy(xs_h.at[idxs_v.at[j]], vals_v, sem).wait()
            pltpu.sync_copy(vals_v, acc.at[idxd_v.at[j]], add=True)
        return 0

    lax.fori_loop(0, ng, grp, 0)
    plsc.subcore_barrier()
    pltpu.sync_copy(acc.at[pl.ds(sid * _SLC, _SLC)],
                    out.at[cid, pl.ds(sid * _SLC, _SLC)])


def _scalar_scatter(src2d, dst2d, xs_flat):
    f = pl.kernel(
        _s_body,
        out_type=jax.ShapeDtypeStruct((NC, NP), jnp.float32),
        mesh=plsc.VectorSubcoreMesh(**_MESH),
        scratch_types=[
            pltpu.VMEM_SHARED((NP,), jnp.float32),
            pltpu.VMEM((4, 128), jnp.int32),
            pltpu.VMEM((4, 128), jnp.int32),
            pltpu.VMEM((128,), jnp.float32),
            pltpu.VMEM((128,), jnp.float32),
            pltpu.SemaphoreType.DMA,
        ],
    )
    return f(src2d, dst2d, xs_flat)


_RR = 6272
_NPASS = NP // (_RR * NC)
_RSLC = _RR // NS
_CAP = 6912


def _make_spmm(nparts):
    def body(src2d, dst2d, *rest):
        tables = rest[:nparts]
        outs = rest[nparts:2 * nparts]
        (acc, srcb, dstb, csrc, cdst, idx2, rows_v, zbuf, sem) = rest[2 * nparts:]
        cid = lax.axis_index("c")
        sid = lax.axis_index("s")
        gstart = sid * _TBASE + jnp.minimum(sid, _TREM)
        ng = _TBASE + jnp.where(sid < _TREM, 1, 0)
        lanes = lax.iota(jnp.int32, 16)
        _zero_vmem_rows(zbuf, 8, 128)

        def one_pass(p, _):
            rng = p * NC + cid
            lo = rng * _RR

            def grp(g, off):
                r0 = (gstart + g) * 4
                pltpu.sync_copy(src2d.at[pl.ds(r0, 4)], srcb)
                pltpu.sync_copy(dst2d.at[pl.ds(r0, 4)], dstb)
                for j in range(4):
                    for q in range(8):
                        d16 = dstb[j, pl.ds(q * 16, 16)]
                        s16 = srcb[j, pl.ds(q * 16, 16)]
                        m = (d16 >= lo) & (d16 < lo + _RR)
                        msum = plsc.cumsum(jnp.where(m, 1, 0))
                        dest = off + msum - 1
                        plsc.store_scatter(csrc, [dest], s16, mask=m)
                        plsc.store_scatter(cdst, [dest], d16 - lo, mask=m)
                        off = off + msum[15]
                return off

            off = lax.fori_loop(0, ng, grp, jnp.int32(0))
            padsrc = (sid * 1237 + lanes * 61) % N
            paddst = _RR + lanes
            for q in range(8):
                plsc.store_scatter(csrc, [off + q * 16 + lanes], padsrc)
                plsc.store_scatter(cdst, [off + q * 16 + lanes], paddst)
            nch = (off + 127) // 128

            for t in range(nparts):
                for q in range(_RSLC // 8):
                    pltpu.sync_copy(zbuf, acc.at[pl.ds(sid * _RSLC + q * 8, 8)])

                @pl.when(sid == 0)
                def _():
                    pltpu.sync_copy(zbuf, acc.at[pl.ds(_RR, 8)])
                    pltpu.sync_copy(zbuf, acc.at[pl.ds(_RR + 8, 8)])
                plsc.subcore_barrier()

                def chunk(c, _, _t=t):
                    for q in range(8):
                        idx2[0, pl.ds(q * 16, 16)] = cdst[pl.ds(c * 128 + q * 16, 16)]
                    pltpu.async_copy(
                        tables[_t].at[csrc.at[pl.ds(c * 128, 128)]],
                        rows_v, sem).wait()
                    pltpu.sync_copy(rows_v, acc.at[idx2.at[0]], add=True)
                    return 0

                lax.fori_loop(0, nch, chunk, 0)
                plsc.subcore_barrier()
                pltpu.sync_copy(
                    acc.at[pl.ds(sid * _RSLC, _RSLC)],
                    outs[t].at[pl.ds(lo + sid * _RSLC, _RSLC)])
                plsc.subcore_barrier()
            return 0

        lax.fori_loop(0, _NPASS, one_pass, 0)

    f = pl.kernel(
        body,
        out_type=[jax.ShapeDtypeStruct((NP, 128), jnp.float32)] * nparts,
        mesh=plsc.VectorSubcoreMesh(**_MESH),
        compiler_params=pltpu.CompilerParams(needs_layout_passes=False),
        scratch_types=[
            pltpu.VMEM_SHARED((_RR + 16, 128), jnp.float32),
            pltpu.VMEM((4, 128), jnp.int32),
            pltpu.VMEM((4, 128), jnp.int32),
            pltpu.VMEM((_CAP,), jnp.int32),
            pltpu.VMEM((_CAP,), jnp.int32),
            pltpu.VMEM((1, 128), jnp.int32),
            pltpu.VMEM((128, 128), jnp.float32),
            pltpu.VMEM((8, 128), jnp.float32),
            pltpu.SemaphoreType.DMA,
        ],
    )
    return f


def _row_scatter128(src2d, dst2d, table):
    return _make_spmm(1)(src2d, dst2d, table)[0]


def _row_scatter256(src2d, dst2d, t0, t1):
    return _make_spmm(2)(src2d, dst2d, t0, t1)


def kernel(x, edge_index, batch, W1, b1, W2, b2, W3, b3, fc1_W, fc1_b, fc2_W, fc2_b):
    src2d = edge_index[0].reshape(EROWS, 128)
    dst2d = edge_index[1].reshape(EROWS, 128)
    x_p = jnp.pad(x, ((0, NP - N), (0, 0)))
    batch_p = jnp.pad(batch, (0, NP - N), constant_values=G)[:, None]

    degp = _deg_scatter(dst2d)
    dis, xs = _stage_a(degp[:, :, None], x_p)
    sp = _scalar_scatter(src2d, dst2d, xs.reshape(NP))
    xw2p = _stage_b(dis, sp[:, :, None], xs, W1, b1[None, :], W2)
    agg2 = _row_scatter128(src2d, dst2d, xw2p)
    xw3pa, xw3pb = _stage_c(dis, agg2, xw2p, b2[None, :], W3)
    agg3a, agg3b = _row_scatter256(src2d, dst2d, xw3pa, xw3pb)
    out = _stage_d(dis, agg3a, agg3b, xw3pa, xw3pb, b3[None, :], batch_p,
                   fc1_W, fc1_b[None, :], fc2_W, fc2_b[None, :])
    return out


# speedup vs baseline: 6.8418x; 1.1361x over previous
import functools

import jax
import jax.numpy as jnp
from jax import lax
from jax.experimental import pallas as pl
from jax.experimental.pallas import tpu as pltpu
from jax.experimental.pallas import tpu_sc as plsc

N = 100000
E = 1600000
G = 128
NP = 100352
BLK = 3584
NBLK = NP // BLK

NC = 2
NS = 16
NW = NC * NS
EROWS = E // 128
_MESH = dict(core_axis_name="c", subcore_axis_name="s",
             num_cores=NC, num_subcores=NS)


def _stage_a_body(degp_ref, x_ref, dis_ref, xs_ref):
    degp = degp_ref[...]
    deg = degp[0] + degp[1] + 1.0
    dis = lax.rsqrt(deg)
    dis_ref[...] = dis
    xs_ref[...] = dis * x_ref[...]


def _stage_a(degp, x_p):
    return pl.pallas_call(
        _stage_a_body,
        grid=(NBLK,),
        in_specs=[pl.BlockSpec((2, BLK, 1), lambda i: (0, i, 0)),
                  pl.BlockSpec((BLK, 1), lambda i: (i, 0))],
        out_specs=[pl.BlockSpec((BLK, 1), lambda i: (i, 0)),
                   pl.BlockSpec((BLK, 1), lambda i: (i, 0))],
        out_shape=[jax.ShapeDtypeStruct((NP, 1), jnp.float32),
                   jax.ShapeDtypeStruct((NP, 1), jnp.float32)],
    )(degp, x_p)


def _stage_b_body(dis_ref, sp_ref, xs_ref, w1_ref, b1_ref, w2_ref, out_ref):
    dis = dis_ref[...]
    sp = sp_ref[...]
    s = sp[0] + sp[1]
    t1 = (dis * (s + xs_ref[...])) * w1_ref[...] + b1_ref[...]
    h1 = jnp.maximum(t1, 0.0)
    xw2 = jnp.dot(h1, w2_ref[...], preferred_element_type=jnp.float32)
    out_ref[...] = xw2 * dis


def _stage_b(dis, sp, xs, W1, b1, W2):
    return pl.pallas_call(
        _stage_b_body,
        grid=(NBLK,),
        in_specs=[pl.BlockSpec((BLK, 1), lambda i: (i, 0)),
                  pl.BlockSpec((2, BLK, 1), lambda i: (0, i, 0)),
                  pl.BlockSpec((BLK, 1), lambda i: (i, 0)),
                  pl.BlockSpec((1, 128), lambda i: (0, 0)),
                  pl.BlockSpec((1, 128), lambda i: (0, 0)),
                  pl.BlockSpec((128, 128), lambda i: (0, 0))],
        out_specs=pl.BlockSpec((BLK, 128), lambda i: (i, 0)),
        out_shape=jax.ShapeDtypeStruct((NP, 128), jnp.float32),
    )(dis, sp, xs, W1, b1, W2)


def _stage_c_body(dis_ref, agg_ref, xwp_ref, b_ref, w_ref, o0_ref, o1_ref):
    dis = dis_ref[...]
    h = jnp.maximum(dis * (agg_ref[...] + xwp_ref[...]) + b_ref[...], 0.0)
    xw = jnp.dot(h, w_ref[...], preferred_element_type=jnp.float32)
    xw = xw * dis
    o0_ref[...] = xw[:, :128]
    o1_ref[...] = xw[:, 128:]


def _stage_c(dis, agg2, xw2p, b2, W3):
    return pl.pallas_call(
        _stage_c_body,
        grid=(NBLK,),
        in_specs=[pl.BlockSpec((BLK, 1), lambda i: (i, 0)),
                  pl.BlockSpec((BLK, 128), lambda i: (i, 0)),
                  pl.BlockSpec((BLK, 128), lambda i: (i, 0)),
                  pl.BlockSpec((1, 128), lambda i: (0, 0)),
                  pl.BlockSpec((128, 256), lambda i: (0, 0))],
        out_specs=[pl.BlockSpec((BLK, 128), lambda i: (i, 0)),
                   pl.BlockSpec((BLK, 128), lambda i: (i, 0))],
        out_shape=[jax.ShapeDtypeStruct((NP, 128), jnp.float32),
                   jax.ShapeDtypeStruct((NP, 128), jnp.float32)],
    )(dis, agg2, xw2p, b2, W3)


def _stage_d_body(dis_ref, a0_ref, a1_ref, x0_ref, x1_ref, b3_ref, batch_ref,
                  fc1w_ref, fc1b_ref, fc2w_ref, fc2b_ref, out_ref, acc_ref):
    i = pl.program_id(0)

    @pl.when(i == 0)
    def _init():
        acc_ref[...] = jnp.zeros_like(acc_ref)

    dis = dis_ref[...]
    agg = jnp.concatenate([a0_ref[...], a1_ref[...]], axis=1)
    xwp = jnp.concatenate([x0_ref[...], x1_ref[...]], axis=1)
    h3 = jnp.maximum(dis * (agg + xwp) + b3_ref[...], 0.0)
    bvec = batch_ref[...]
    g_lo = bvec[0, 0]
    g_hi = bvec[BLK - 1, 0]
    gids = lax.broadcasted_iota(jnp.int32, (G, 1), 0)

    def seg_body(g, _):
        m = bvec == g
        contrib = jnp.max(jnp.where(m, h3, 0.0), axis=0, keepdims=True)
        acc_ref[...] = jnp.maximum(acc_ref[...],
                                   jnp.where(gids == g, contrib, 0.0))
        return 0

    lax.fori_loop(g_lo, g_hi + 1, seg_body, 0)

    @pl.when(i == NBLK - 1)
    def _fin():
        p = acc_ref[...]
        z = jnp.maximum(
            jnp.dot(p, fc1w_ref[...], preferred_element_type=jnp.float32)
            + fc1b_ref[...], 0.0)
        out_ref[...] = jnp.maximum(
            jnp.dot(z, fc2w_ref[...], preferred_element_type=jnp.float32)
            + fc2b_ref[...], 0.0)


def _stage_d(dis, a0, a1, x0, x1, b3, batch_p, fc1_W, fc1_b, fc2_W, fc2_b):
    return pl.pallas_call(
        _stage_d_body,
        grid=(NBLK,),
        in_specs=[pl.BlockSpec((BLK, 1), lambda i: (i, 0)),
                  pl.BlockSpec((BLK, 128), lambda i: (i, 0)),
                  pl.BlockSpec((BLK, 128), lambda i: (i, 0)),
                  pl.BlockSpec((BLK, 128), lambda i: (i, 0)),
                  pl.BlockSpec((BLK, 128), lambda i: (i, 0)),
                  pl.BlockSpec((1, 256), lambda i: (0, 0)),
                  pl.BlockSpec((BLK, 1), lambda i: (i, 0)),
                  pl.BlockSpec((256, 128), lambda i: (0, 0)),
                  pl.BlockSpec((1, 128), lambda i: (0, 0)),
                  pl.BlockSpec((128, 10), lambda i: (0, 0)),
                  pl.BlockSpec((1, 10), lambda i: (0, 0))],
        out_specs=pl.BlockSpec((G, 10), lambda i: (0, 0)),
        out_shape=jax.ShapeDtypeStruct((G, 10), jnp.float32),
        scratch_shapes=[pltpu.VMEM((G, 256), jnp.float32)],
    )(dis, a0, a1, x0, x1, b3, batch_p, fc1_W, fc1_b, fc2_W, fc2_b)


_NG = EROWS // 4
_GBASE, _GREM = _NG // NW, _NG % NW
_NG_SC = _NG
_TBASE, _TREM = _NG // NS, _NG % NS

_SLC = NP // NS


def _zero_vmem_rows(buf, rows, f):
    zero = jnp.zeros((16,), jnp.float32)
    for r in range(rows):
        for q in range(f // 16):
            buf[r, pl.ds(q * 16, 16)] = zero


def _deg_body(dst2d, out, acc, idx_v, ones_v, zbuf):
    cid = lax.axis_index("c")
    sid = lax.axis_index("s")
    wid = sid * NC + cid
    for q in range(8):
        zbuf[pl.ds(q * 16, 16)] = jnp.zeros((16,), jnp.float32)
        ones_v[pl.ds(q * 16, 16)] = jnp.ones((16,), jnp.float32)
    for q in range(_SLC // 128):
        pltpu.sync_copy(zbuf, acc.at[pl.ds(sid * _SLC + q * 128, 128)])
    plsc.subcore_barrier()
    gstart = wid * _GBASE + jnp.minimum(wid, _GREM)
    ng = _GBASE + (wid < _GREM).astype(jnp.int32)

    def grp(g, _):
        r0 = (gstart + g) * 4
        pltpu.sync_copy(dst2d.at[pl.ds(r0, 4)], idx_v)
        for j in range(4):
            pltpu.sync_copy(ones_v, acc.at[idx_v.at[j]], add=True)
        return 0

    lax.fori_loop(0, ng, grp, 0)
    plsc.subcore_barrier()
    pltpu.sync_copy(acc.at[pl.ds(sid * _SLC, _SLC)],
                    out.at[cid, pl.ds(sid * _SLC, _SLC)])


def _deg_scatter(dst2d):
    f = pl.kernel(
        _deg_body,
        out_type=jax.ShapeDtypeStruct((NC, NP), jnp.float32),
        mesh=plsc.VectorSubcoreMesh(**_MESH),
        scratch_types=[
            pltpu.VMEM_SHARED((NP,), jnp.float32),
            pltpu.VMEM((4, 128), jnp.int32),
            pltpu.VMEM((128,), jnp.float32),
            pltpu.VMEM((128,), jnp.float32),
        ],
    )
    return f(dst2d)


def _s_body(src2d, dst2d, xs_h, out, acc, idxs_v, idxd_v, vals_v, zbuf, sem):
    cid = lax.axis_index("c")
    sid = lax.axis_index("s")
    wid = sid * NC + cid
    for q in range(8):
        zbuf[pl.ds(q * 16, 16)] = jnp.zeros((16,), jnp.float32)
    for q in range(_SLC // 128):
        pltpu.sync_copy(zbuf, acc.at[pl.ds(sid * _SLC + q * 128, 128)])
    plsc.subcore_barrier()
    gstart = wid * _GBASE + jnp.minimum(wid, _GREM)
    ng = _GBASE + (wid < _GREM).astype(jnp.int32)

    def grp(g, _):
        r0 = (gstart + g) * 4
        pltpu.sync_copy(src2d.at[pl.ds(r0, 4)], idxs_v)
        pltpu.sync_copy(dst2d.at[pl.ds(r0, 4)], idxd_v)
        for j in range(4):
            pltpu.async_copy(xs_h.at[idxs_v.at[j]], vals_v, sem).wait()
            pltpu.sync_copy(vals_v, acc.at[idxd_v.at[j]], add=True)
        return 0

    lax.fori_loop(0, ng, grp, 0)
    plsc.subcore_barrier()
    pltpu.sync_copy(acc.at[pl.ds(sid * _SLC, _SLC)],
                    out.at[cid, pl.ds(sid * _SLC, _SLC)])


def _scalar_scatter(src2d, dst2d, xs_flat):
    f = pl.kernel(
        _s_body,
        out_type=jax.ShapeDtypeStruct((NC, NP), jnp.float32),
        mesh=plsc.VectorSubcoreMesh(**_MESH),
        scratch_types=[
            pltpu.VMEM_SHARED((NP,), jnp.float32),
            pltpu.VMEM((4, 128), jnp.int32),
            pltpu.VMEM((4, 128), jnp.int32),
            pltpu.VMEM((128,), jnp.float32),
            pltpu.VMEM((128,), jnp.float32),
            pltpu.SemaphoreType.DMA,
        ],
    )
    return f(src2d, dst2d, xs_flat)


_RR = 6272
_NPASS = NP // (_RR * NC)
_RSLC = _RR // NS
_CAP = 7168


def _make_spmm(nparts):
    def body(src2d, dst2d, *rest):
        tables = rest[:nparts]
        outs = rest[nparts:2 * nparts]
        (acc, srcb, dstb, csrc, cdst, idx2, rows0, rows1, zbuf,
         sem0, sem1) = rest[2 * nparts:]
        cid = lax.axis_index("c")
        sid = lax.axis_index("s")
        gstart = sid * _TBASE + jnp.minimum(sid, _TREM)
        ng = _TBASE + jnp.where(sid < _TREM, 1, 0)
        lanes = lax.iota(jnp.int32, 16)
        _zero_vmem_rows(zbuf, 8, 128)

        def one_pass(p, _):
            rng = p * NC + cid
            lo = rng * _RR

            def grp(g, off):
                r0 = (gstart + g) * 4
                pltpu.sync_copy(src2d.at[pl.ds(r0, 4)], srcb)
                pltpu.sync_copy(dst2d.at[pl.ds(r0, 4)], dstb)
                for j in range(4):
                    for q in range(8):
                        d16 = dstb[j, pl.ds(q * 16, 16)]
                        s16 = srcb[j, pl.ds(q * 16, 16)]
                        m = (d16 >= lo) & (d16 < lo + _RR)
                        msum = plsc.cumsum(jnp.where(m, 1, 0))
                        dest = off + msum - 1
                        plsc.store_scatter(csrc, [dest], s16, mask=m)
                        plsc.store_scatter(cdst, [dest], d16 - lo, mask=m)
                        off = off + msum[15]
                return off

            off = lax.fori_loop(0, ng, grp, jnp.int32(0))
            padsrc = (sid * 1237 + lanes * 61) % N
            paddst = _RR + lanes
            for q in range(16):
                plsc.store_scatter(csrc, [off + q * 16 + lanes], padsrc)
                plsc.store_scatter(cdst, [off + q * 16 + lanes], paddst)
            nch2 = jnp.maximum((off + 255) // 256, 1)
            last = 2 * nch2 - 1

            for t in range(nparts):
                for q in range(_RSLC // 8):
                    pltpu.sync_copy(zbuf, acc.at[pl.ds(sid * _RSLC + q * 8, 8)])

                @pl.when(sid == 0)
                def _():
                    pltpu.sync_copy(zbuf, acc.at[pl.ds(_RR, 8)])
                    pltpu.sync_copy(zbuf, acc.at[pl.ds(_RR + 8, 8)])
                plsc.subcore_barrier()

                def _gat(c, buf, sem_, _t=t):
                    return pltpu.async_copy(
                        tables[_t].at[csrc.at[pl.ds(c * 128, 128)]],
                        buf, sem_)

                _gat(0, rows0, sem0)

                def chunk2(k, _, _t=t):
                    c0 = 2 * k
                    _gat(c0 + 1, rows1, sem1)
                    pltpu.make_async_copy(
                        tables[_t].at[csrc.at[pl.ds(0, 128)]],
                        rows0, sem0).wait()
                    for q in range(8):
                        idx2[0, pl.ds(q * 16, 16)] = cdst[pl.ds(c0 * 128 + q * 16, 16)]
                    pltpu.sync_copy(rows0, acc.at[idx2.at[0]], add=True)
                    _gat(jnp.minimum(c0 + 2, last), rows0, sem0)
                    pltpu.make_async_copy(
                        tables[_t].at[csrc.at[pl.ds(0, 128)]],
                        rows1, sem1).wait()
                    for q in range(8):
                        idx2[1, pl.ds(q * 16, 16)] = cdst[pl.ds((c0 + 1) * 128 + q * 16, 16)]
                    pltpu.sync_copy(rows1, acc.at[idx2.at[1]], add=True)
                    return 0

                lax.fori_loop(0, nch2, chunk2, 0)
                pltpu.make_async_copy(
                    tables[t].at[csrc.at[pl.ds(0, 128)]], rows0, sem0).wait()
                plsc.subcore_barrier()
                pltpu.sync_copy(
                    acc.at[pl.ds(sid * _RSLC, _RSLC)],
                    outs[t].at[pl.ds(lo + sid * _RSLC, _RSLC)])
                plsc.subcore_barrier()
            return 0

        lax.fori_loop(0, _NPASS, one_pass, 0)

    f = pl.kernel(
        body,
        out_type=[jax.ShapeDtypeStruct((NP, 128), jnp.float32)] * nparts,
        mesh=plsc.VectorSubcoreMesh(**_MESH),
        compiler_params=pltpu.CompilerParams(needs_layout_passes=False),
        scratch_types=[
            pltpu.VMEM_SHARED((_RR + 16, 128), jnp.float32),
            pltpu.VMEM((4, 128), jnp.int32),
            pltpu.VMEM((4, 128), jnp.int32),
            pltpu.VMEM((_CAP,), jnp.int32),
            pltpu.VMEM((_CAP,), jnp.int32),
            pltpu.VMEM((2, 128), jnp.int32),
            pltpu.VMEM((128, 128), jnp.float32),
            pltpu.VMEM((128, 128), jnp.float32),
            pltpu.VMEM((8, 128), jnp.float32),
            pltpu.SemaphoreType.DMA,
            pltpu.SemaphoreType.DMA,
        ],
    )
    return f


def _row_scatter128(src2d, dst2d, table):
    return _make_spmm(1)(src2d, dst2d, table)[0]


def _row_scatter256(src2d, dst2d, t0, t1):
    return _make_spmm(2)(src2d, dst2d, t0, t1)


def kernel(x, edge_index, batch, W1, b1, W2, b2, W3, b3, fc1_W, fc1_b, fc2_W, fc2_b):
    src2d = edge_index[0].reshape(EROWS, 128)
    dst2d = edge_index[1].reshape(EROWS, 128)
    x_p = jnp.pad(x, ((0, NP - N), (0, 0)))
    batch_p = jnp.pad(batch, (0, NP - N), constant_values=G)[:, None]

    degp = _deg_scatter(dst2d)
    dis, xs = _stage_a(degp[:, :, None], x_p)
    sp = _scalar_scatter(src2d, dst2d, xs.reshape(NP))
    xw2p = _stage_b(dis, sp[:, :, None], xs, W1, b1[None, :], W2)
    agg2 = _row_scatter128(src2d, dst2d, xw2p)
    xw3pa, xw3pb = _stage_c(dis, agg2, xw2p, b2[None, :], W3)
    agg3a, agg3b = _row_scatter256(src2d, dst2d, xw3pa, xw3pb)
    out = _stage_d(dis, agg3a, agg3b, xw3pa, xw3pb, b3[None, :], batch_p,
                   fc1_W, fc1_b[None, :], fc2_W, fc2_b[None, :])
    return out


# speedup vs baseline: 11.5140x; 1.6829x over previous
import functools

import jax
import jax.numpy as jnp
from jax import lax
from jax.experimental import pallas as pl
from jax.experimental.pallas import tpu as pltpu
from jax.experimental.pallas import tpu_sc as plsc

N = 100000
E = 1600000
G = 128
NP = 100352
BLK = 3584
NBLK = NP // BLK

NC = 2
NS = 16
NW = NC * NS
EROWS = E // 128
_MESH = dict(core_axis_name="c", subcore_axis_name="s",
             num_cores=NC, num_subcores=NS)


def _stage_a_body(degp_ref, x_ref, dis_ref, xs_ref):
    degp = degp_ref[...]
    deg = degp[0] + degp[1] + 1.0
    dis = lax.rsqrt(deg)
    dis_ref[...] = dis
    xs_ref[...] = dis * x_ref[...]


def _stage_a(degp, x_p):
    return pl.pallas_call(
        _stage_a_body,
        grid=(NBLK,),
        in_specs=[pl.BlockSpec((2, BLK, 1), lambda i: (0, i, 0)),
                  pl.BlockSpec((BLK, 1), lambda i: (i, 0))],
        out_specs=[pl.BlockSpec((BLK, 1), lambda i: (i, 0)),
                   pl.BlockSpec((BLK, 1), lambda i: (i, 0))],
        out_shape=[jax.ShapeDtypeStruct((NP, 1), jnp.float32),
                   jax.ShapeDtypeStruct((NP, 1), jnp.float32)],
    )(degp, x_p)


def _stage_b_body(dis_ref, sp_ref, xs_ref, w1_ref, b1_ref, w2_ref, out_ref):
    dis = dis_ref[...]
    sp = sp_ref[...]
    s = sp[0] + sp[1]
    t1 = (dis * (s + xs_ref[...])) * w1_ref[...] + b1_ref[...]
    h1 = jnp.maximum(t1, 0.0)
    xw2 = jnp.dot(h1, w2_ref[...], preferred_element_type=jnp.float32)
    out_ref[...] = xw2 * dis


def _stage_b(dis, sp, xs, W1, b1, W2):
    return pl.pallas_call(
        _stage_b_body,
        grid=(NBLK,),
        in_specs=[pl.BlockSpec((BLK, 1), lambda i: (i, 0)),
                  pl.BlockSpec((2, BLK, 1), lambda i: (0, i, 0)),
                  pl.BlockSpec((BLK, 1), lambda i: (i, 0)),
                  pl.BlockSpec((1, 128), lambda i: (0, 0)),
                  pl.BlockSpec((1, 128), lambda i: (0, 0)),
                  pl.BlockSpec((128, 128), lambda i: (0, 0))],
        out_specs=pl.BlockSpec((BLK, 128), lambda i: (i, 0)),
        out_shape=jax.ShapeDtypeStruct((NP, 128), jnp.float32),
    )(dis, sp, xs, W1, b1, W2)


def _stage_c_body(dis_ref, agg_ref, xwp_ref, b_ref, w_ref, o0_ref, o1_ref):
    dis = dis_ref[...]
    h = jnp.maximum(dis * (agg_ref[...] + xwp_ref[...]) + b_ref[...], 0.0)
    xw = jnp.dot(h, w_ref[...], preferred_element_type=jnp.float32)
    xw = xw * dis
    o0_ref[...] = xw[:, :128]
    o1_ref[...] = xw[:, 128:]


def _stage_c(dis, agg2, xw2p, b2, W3):
    return pl.pallas_call(
        _stage_c_body,
        grid=(NBLK,),
        in_specs=[pl.BlockSpec((BLK, 1), lambda i: (i, 0)),
                  pl.BlockSpec((BLK, 128), lambda i: (i, 0)),
                  pl.BlockSpec((BLK, 128), lambda i: (i, 0)),
                  pl.BlockSpec((1, 128), lambda i: (0, 0)),
                  pl.BlockSpec((128, 256), lambda i: (0, 0))],
        out_specs=[pl.BlockSpec((BLK, 128), lambda i: (i, 0)),
                   pl.BlockSpec((BLK, 128), lambda i: (i, 0))],
        out_shape=[jax.ShapeDtypeStruct((NP, 128), jnp.float32),
                   jax.ShapeDtypeStruct((NP, 128), jnp.float32)],
    )(dis, agg2, xw2p, b2, W3)


def _stage_d_body(dis_ref, a0_ref, a1_ref, x0_ref, x1_ref, b3_ref, batch_ref,
                  fc1w_ref, fc1b_ref, fc2w_ref, fc2b_ref, out_ref, acc_ref):
    i = pl.program_id(0)

    @pl.when(i == 0)
    def _init():
        acc_ref[...] = jnp.zeros_like(acc_ref)

    dis = dis_ref[...]
    agg = jnp.concatenate([a0_ref[...], a1_ref[...]], axis=1)
    xwp = jnp.concatenate([x0_ref[...], x1_ref[...]], axis=1)
    h3 = jnp.maximum(dis * (agg + xwp) + b3_ref[...], 0.0)
    bvec = batch_ref[...]
    g_lo = bvec[0, 0]
    g_hi = bvec[BLK - 1, 0]
    gids = lax.broadcasted_iota(jnp.int32, (G, 1), 0)

    def seg_body(g, _):
        m = bvec == g
        contrib = jnp.max(jnp.where(m, h3, 0.0), axis=0, keepdims=True)
        acc_ref[...] = jnp.maximum(acc_ref[...],
                                   jnp.where(gids == g, contrib, 0.0))
        return 0

    lax.fori_loop(g_lo, g_hi + 1, seg_body, 0)

    @pl.when(i == NBLK - 1)
    def _fin():
        p = acc_ref[...]
        z = jnp.maximum(
            jnp.dot(p, fc1w_ref[...], preferred_element_type=jnp.float32)
            + fc1b_ref[...], 0.0)
        out_ref[...] = jnp.maximum(
            jnp.dot(z, fc2w_ref[...], preferred_element_type=jnp.float32)
            + fc2b_ref[...], 0.0)


def _stage_d(dis, a0, a1, x0, x1, b3, batch_p, fc1_W, fc1_b, fc2_W, fc2_b):
    return pl.pallas_call(
        _stage_d_body,
        grid=(NBLK,),
        in_specs=[pl.BlockSpec((BLK, 1), lambda i: (i, 0)),
                  pl.BlockSpec((BLK, 128), lambda i: (i, 0)),
                  pl.BlockSpec((BLK, 128), lambda i: (i, 0)),
                  pl.BlockSpec((BLK, 128), lambda i: (i, 0)),
                  pl.BlockSpec((BLK, 128), lambda i: (i, 0)),
                  pl.BlockSpec((1, 256), lambda i: (0, 0)),
                  pl.BlockSpec((BLK, 1), lambda i: (i, 0)),
                  pl.BlockSpec((256, 128), lambda i: (0, 0)),
                  pl.BlockSpec((1, 128), lambda i: (0, 0)),
                  pl.BlockSpec((128, 10), lambda i: (0, 0)),
                  pl.BlockSpec((1, 10), lambda i: (0, 0))],
        out_specs=pl.BlockSpec((G, 10), lambda i: (0, 0)),
        out_shape=jax.ShapeDtypeStruct((G, 10), jnp.float32),
        scratch_shapes=[pltpu.VMEM((G, 256), jnp.float32)],
    )(dis, a0, a1, x0, x1, b3, batch_p, fc1_W, fc1_b, fc2_W, fc2_b)


_NG = EROWS // 4
_GBASE, _GREM = _NG // NW, _NG % NW
_NG_SC = _NG
_TBASE, _TREM = _NG // NS, _NG % NS

_SLC = NP // NS


def _zero_vmem_rows(buf, rows, f):
    zero = jnp.zeros((16,), jnp.float32)
    for r in range(rows):
        for q in range(f // 16):
            buf[r, pl.ds(q * 16, 16)] = zero


def _deg_body(dst2d, out, acc, idx_v, ones_v, zbuf):
    cid = lax.axis_index("c")
    sid = lax.axis_index("s")
    wid = sid * NC + cid
    for q in range(8):
        zbuf[pl.ds(q * 16, 16)] = jnp.zeros((16,), jnp.float32)
        ones_v[pl.ds(q * 16, 16)] = jnp.ones((16,), jnp.float32)
    for q in range(_SLC // 128):
        pltpu.sync_copy(zbuf, acc.at[pl.ds(sid * _SLC + q * 128, 128)])
    plsc.subcore_barrier()
    gstart = wid * _GBASE + jnp.minimum(wid, _GREM)
    ng = _GBASE + (wid < _GREM).astype(jnp.int32)

    def grp(g, _):
        r0 = (gstart + g) * 4
        pltpu.sync_copy(dst2d.at[pl.ds(r0, 4)], idx_v)
        for j in range(4):
            pltpu.sync_copy(ones_v, acc.at[idx_v.at[j]], add=True)
        return 0

    lax.fori_loop(0, ng, grp, 0)
    plsc.subcore_barrier()
    pltpu.sync_copy(acc.at[pl.ds(sid * _SLC, _SLC)],
                    out.at[cid, pl.ds(sid * _SLC, _SLC)])


def _deg_scatter(dst2d):
    f = pl.kernel(
        _deg_body,
        out_type=jax.ShapeDtypeStruct((NC, NP), jnp.float32),
        mesh=plsc.VectorSubcoreMesh(**_MESH),
        scratch_types=[
            pltpu.VMEM_SHARED((NP,), jnp.float32),
            pltpu.VMEM((4, 128), jnp.int32),
            pltpu.VMEM((128,), jnp.float32),
            pltpu.VMEM((128,), jnp.float32),
        ],
    )
    return f(dst2d)


def _s_body(src2d, dst2d, xs_h, out, acc, idxs_v, idxd_v, vals_v, zbuf, sem):
    cid = lax.axis_index("c")
    sid = lax.axis_index("s")
    wid = sid * NC + cid
    for q in range(8):
        zbuf[pl.ds(q * 16, 16)] = jnp.zeros((16,), jnp.float32)
    for q in range(_SLC // 128):
        pltpu.sync_copy(zbuf, acc.at[pl.ds(sid * _SLC + q * 128, 128)])
    plsc.subcore_barrier()
    gstart = wid * _GBASE + jnp.minimum(wid, _GREM)
    ng = _GBASE + (wid < _GREM).astype(jnp.int32)

    def grp(g, _):
        r0 = (gstart + g) * 4
        pltpu.sync_copy(src2d.at[pl.ds(r0, 4)], idxs_v)
        pltpu.sync_copy(dst2d.at[pl.ds(r0, 4)], idxd_v)
        for j in range(4):
            pltpu.async_copy(xs_h.at[idxs_v.at[j]], vals_v, sem).wait()
            pltpu.sync_copy(vals_v, acc.at[idxd_v.at[j]], add=True)
        return 0

    lax.fori_loop(0, ng, grp, 0)
    plsc.subcore_barrier()
    pltpu.sync_copy(acc.at[pl.ds(sid * _SLC, _SLC)],
                    out.at[cid, pl.ds(sid * _SLC, _SLC)])


def _scalar_scatter(src2d, dst2d, xs_flat):
    f = pl.kernel(
        _s_body,
        out_type=jax.ShapeDtypeStruct((NC, NP), jnp.float32),
        mesh=plsc.VectorSubcoreMesh(**_MESH),
        scratch_types=[
            pltpu.VMEM_SHARED((NP,), jnp.float32),
            pltpu.VMEM((4, 128), jnp.int32),
            pltpu.VMEM((4, 128), jnp.int32),
            pltpu.VMEM((128,), jnp.float32),
            pltpu.VMEM((128,), jnp.float32),
            pltpu.SemaphoreType.DMA,
        ],
    )
    return f(src2d, dst2d, xs_flat)


_RR = 6272
_NPASS = NP // (_RR * NC)
_RSLC = _RR // NS
_CAP = 7168


def _make_spmm(nparts):
    def body(src2d, dst2d, *rest):
        tables = rest[:nparts]
        outs = rest[nparts:2 * nparts]
        (acc, srcb, dstb, srcc, dstc, csrc, cdst, idx2, rows0, rows1, zbuf,
         sem0, sem1, sema, semb) = rest[2 * nparts:]
        cid = lax.axis_index("c")
        sid = lax.axis_index("s")
        gstart = sid * _TBASE + jnp.minimum(sid, _TREM)
        ng = _TBASE + jnp.where(sid < _TREM, 1, 0)
        lanes = lax.iota(jnp.int32, 16)
        _zero_vmem_rows(zbuf, 8, 128)

        def one_pass(p, _):
            rng = p * NC + cid
            lo = rng * _RR

            def eload(g, sbuf, dbuf, sem_):
                r0 = (gstart + g) * 4
                pltpu.async_copy(src2d.at[pl.ds(r0, 4)], sbuf, sem_)
                pltpu.async_copy(dst2d.at[pl.ds(r0, 4)], dbuf, sem_)

            def ewait(sbuf, dbuf, sem_):
                pltpu.make_async_copy(src2d.at[pl.ds(0, 4)], sbuf, sem_).wait()
                pltpu.make_async_copy(dst2d.at[pl.ds(0, 4)], dbuf, sem_).wait()

            def compact(sbuf, dbuf, off, flag=None):
                for j in range(4):
                    for q in range(8):
                        d16 = dbuf[j, pl.ds(q * 16, 16)]
                        s16 = sbuf[j, pl.ds(q * 16, 16)]
                        m = (d16 >= lo) & (d16 < lo + _RR)
                        mi = jnp.where(m, 1, 0) if flag is None \
                            else jnp.where(m, flag, 0)
                        m2 = mi > 0
                        msum = plsc.cumsum(mi)
                        dest = off + msum - 1
                        plsc.store_scatter(csrc, [dest], s16, mask=m2)
                        plsc.store_scatter(cdst, [dest], d16 - lo, mask=m2)
                        off = off + msum[15]
                return off

            npair = ng // 2
            tail = ng - 2 * npair
            eload(0, srcb, dstb, sema)

            def gpair(k, off):
                eload(2 * k + 1, srcc, dstc, semb)
                ewait(srcb, dstb, sema)
                off = compact(srcb, dstb, off)
                eload(jnp.minimum(2 * k + 2, ng - 1), srcb, dstb, sema)
                ewait(srcc, dstc, semb)
                return compact(srcc, dstc, off)

            off = lax.fori_loop(0, npair, gpair, jnp.int32(0))
            ewait(srcb, dstb, sema)
            off = compact(srcb, dstb, off, flag=tail)
            padsrc = (sid * 1237 + lanes * 61) % N
            paddst = _RR + lanes
            for q in range(16):
                plsc.store_scatter(csrc, [off + q * 16 + lanes], padsrc)
                plsc.store_scatter(cdst, [off + q * 16 + lanes], paddst)
            nch2 = jnp.maximum((off + 255) // 256, 1)
            last = 2 * nch2 - 1

            for t in range(nparts):
                for q in range(_RSLC // 8):
                    pltpu.sync_copy(zbuf, acc.at[pl.ds(sid * _RSLC + q * 8, 8)])

                @pl.when(sid == 0)
                def _():
                    pltpu.sync_copy(zbuf, acc.at[pl.ds(_RR, 8)])
                    pltpu.sync_copy(zbuf, acc.at[pl.ds(_RR + 8, 8)])
                plsc.subcore_barrier()

                def _gat(c, buf, sem_, _t=t):
                    return pltpu.async_copy(
                        tables[_t].at[csrc.at[pl.ds(c * 128, 128)]],
                        buf, sem_)

                _gat(0, rows0, sem0)

                def chunk2(k, _, _t=t):
                    c0 = 2 * k
                    _gat(c0 + 1, rows1, sem1)
                    pltpu.make_async_copy(
                        tables[_t].at[csrc.at[pl.ds(0, 128)]],
                        rows0, sem0).wait()
                    for q in range(8):
                        idx2[0, pl.ds(q * 16, 16)] = cdst[pl.ds(c0 * 128 + q * 16, 16)]
                    pltpu.sync_copy(rows0, acc.at[idx2.at[0]], add=True)
                    _gat(jnp.minimum(c0 + 2, last), rows0, sem0)
                    pltpu.make_async_copy(
                        tables[_t].at[csrc.at[pl.ds(0, 128)]],
                        rows1, sem1).wait()
                    for q in range(8):
                        idx2[1, pl.ds(q * 16, 16)] = cdst[pl.ds((c0 + 1) * 128 + q * 16, 16)]
                    pltpu.sync_copy(rows1, acc.at[idx2.at[1]], add=True)
                    return 0

                lax.fori_loop(0, nch2, chunk2, 0)
                pltpu.make_async_copy(
                    tables[t].at[csrc.at[pl.ds(0, 128)]], rows0, sem0).wait()
                plsc.subcore_barrier()
                pltpu.sync_copy(
                    acc.at[pl.ds(sid * _RSLC, _RSLC)],
                    outs[t].at[pl.ds(lo + sid * _RSLC, _RSLC)])
                plsc.subcore_barrier()
            return 0

        lax.fori_loop(0, _NPASS, one_pass, 0)

    f = pl.kernel(
        body,
        out_type=[jax.ShapeDtypeStruct((NP, 128), jnp.float32)] * nparts,
        mesh=plsc.VectorSubcoreMesh(**_MESH),
        compiler_params=pltpu.CompilerParams(needs_layout_passes=False),
        scratch_types=[
            pltpu.VMEM_SHARED((_RR + 16, 128), jnp.float32),
            pltpu.VMEM((4, 128), jnp.int32),
            pltpu.VMEM((4, 128), jnp.int32),
            pltpu.VMEM((4, 128), jnp.int32),
            pltpu.VMEM((4, 128), jnp.int32),
            pltpu.VMEM((_CAP,), jnp.int32),
            pltpu.VMEM((_CAP,), jnp.int32),
            pltpu.VMEM((2, 128), jnp.int32),
            pltpu.VMEM((128, 128), jnp.float32),
            pltpu.VMEM((128, 128), jnp.float32),
            pltpu.VMEM((8, 128), jnp.float32),
            pltpu.SemaphoreType.DMA,
            pltpu.SemaphoreType.DMA,
            pltpu.SemaphoreType.DMA,
            pltpu.SemaphoreType.DMA,
        ],
    )
    return f


def _row_scatter128(src2d, dst2d, table):
    return _make_spmm(1)(src2d, dst2d, table)[0]


def _row_scatter256(src2d, dst2d, t0, t1):
    return _make_spmm(2)(src2d, dst2d, t0, t1)


def kernel(x, edge_index, batch, W1, b1, W2, b2, W3, b3, fc1_W, fc1_b, fc2_W, fc2_b):
    src2d = edge_index[0].reshape(EROWS, 128)
    dst2d = edge_index[1].reshape(EROWS, 128)
    x_p = jnp.pad(x, ((0, NP - N), (0, 0)))
    batch_p = jnp.pad(batch, (0, NP - N), constant_values=G)[:, None]

    degp = _deg_scatter(dst2d)
    dis, xs = _stage_a(degp[:, :, None], x_p)
    sp = _scalar_scatter(src2d, dst2d, xs.reshape(NP))
    xw2p = _stage_b(dis, sp[:, :, None], xs, W1, b1[None, :], W2)
    agg2 = _row_scatter128(src2d, dst2d, xw2p)
    xw3pa, xw3pb = _stage_c(dis, agg2, xw2p, b2[None, :], W3)
    agg3a, agg3b = _row_scatter256(src2d, dst2d, xw3pa, xw3pb)
    out = _stage_d(dis, agg3a, agg3b, xw3pa, xw3pb, b3[None, :], batch_p,
                   fc1_W, fc1_b[None, :], fc2_W, fc2_b[None, :])
    return out
